# baseline (speedup 1.0000x reference)
import functools

import jax
import jax.numpy as jnp
from jax import lax
from jax.experimental import pallas as pl
from jax.experimental.pallas import tpu as pltpu
from jax.experimental.pallas import tpu_sc as plsc


_BM = 256


def _mm_body(a_ref, x_ref, o_ref):
    o_ref[...] = jnp.dot(a_ref[...], x_ref[...],
                         preferred_element_type=jnp.float32)


def _mm(a, x, bm=_BM):
    m, k = a.shape
    n = x.shape[1]
    return pl.pallas_call(
        _mm_body,
        grid=(m // bm,),
        in_specs=[
            pl.BlockSpec((bm, k), lambda i: (i, 0)),
            pl.BlockSpec((k, n), lambda i: (0, 0)),
        ],
        out_specs=pl.BlockSpec((bm, n), lambda i: (i, 0)),
        out_shape=jax.ShapeDtypeStruct((m, n), jnp.float32),
    )(a, x)


def _mm2_body(a_ref, x1_ref, x2_ref, o1_ref, o2_ref):
    a = a_ref[...].astype(jnp.bfloat16)
    o1_ref[...] = jnp.dot(a, x1_ref[...], preferred_element_type=jnp.float32)
    o2_ref[...] = jnp.dot(a, x2_ref[...], preferred_element_type=jnp.float32)


def _mm2(a, x1, x2, bm=_BM):
    m, k = a.shape
    n = x1.shape[1]
    return pl.pallas_call(
        _mm2_body,
        grid=(m // bm,),
        in_specs=[
            pl.BlockSpec((bm, k), lambda i: (i, 0)),
            pl.BlockSpec((k, n), lambda i: (0, 0)),
            pl.BlockSpec((k, n), lambda i: (0, 0)),
        ],
        out_specs=(
            pl.BlockSpec((bm, n), lambda i: (i, 0)),
            pl.BlockSpec((bm, n), lambda i: (i, 0)),
        ),
        out_shape=(
            jax.ShapeDtypeStruct((m, n), jnp.float32),
            jax.ShapeDtypeStruct((m, n), jnp.float32),
        ),
    )(a, x1, x2)


def _mm_users_body(a_ref, x_ref, ue_ref, g1_ref, g2_ref, js_ref, o_ref):
    g3 = jnp.dot(a_ref[...], x_ref[...], preferred_element_type=jnp.float32)
    o_ref[...] = (0.25 * (ue_ref[...] + g1_ref[...] + g2_ref[...])
                  + g3 * js_ref[...])


def _mm_users(a, x, ue, g1, g2, js, bm=_BM):
    m, k = a.shape
    n = x.shape[1]
    return pl.pallas_call(
        _mm_users_body,
        grid=(m // bm,),
        in_specs=[
            pl.BlockSpec((bm, k), lambda i: (i, 0)),
            pl.BlockSpec((k, n), lambda i: (0, 0)),
            pl.BlockSpec((bm, n), lambda i: (i, 0)),
            pl.BlockSpec((bm, n), lambda i: (i, 0)),
            pl.BlockSpec((bm, n), lambda i: (i, 0)),
            pl.BlockSpec((bm, 1), lambda i: (i, 0)),
        ],
        out_specs=pl.BlockSpec((bm, n), lambda i: (i, 0)),
        out_shape=jax.ShapeDtypeStruct((m, n), jnp.float32),
    )(a, x, ue, g1, g2, js)


def _mm_items_body(a_ref, x_ref, ie_ref, g1_ref, g2_ref, o_ref):
    g3 = jnp.dot(a_ref[...], x_ref[...], preferred_element_type=jnp.float32)
    o_ref[...] = 0.25 * (ie_ref[...] + g1_ref[...] + g2_ref[...] + g3)


def _mm_items(a, x, ie, g1, g2, bm=_BM):
    m, k = a.shape
    n = x.shape[1]
    return pl.pallas_call(
        _mm_items_body,
        grid=(m // bm,),
        in_specs=[
            pl.BlockSpec((bm, k), lambda i: (i, 0)),
            pl.BlockSpec((k, n), lambda i: (0, 0)),
            pl.BlockSpec((bm, n), lambda i: (i, 0)),
            pl.BlockSpec((bm, n), lambda i: (i, 0)),
            pl.BlockSpec((bm, n), lambda i: (i, 0)),
        ],
        out_specs=pl.BlockSpec((bm, n), lambda i: (i, 0)),
        out_shape=jax.ShapeDtypeStruct((m, n), jnp.float32),
    )(a, x, ie, g1, g2)


def _sc_gather(utab, itab, idx3d):
    _, f = utab.shape
    nw, three, lw = idx3d.shape
    info = plsc.get_sparse_core_info()
    assert nw == info.num_cores * info.num_subcores and three == 3
    b = nw * lw
    mesh = plsc.VectorSubcoreMesh(core_axis_name="c", subcore_axis_name="s")

    @functools.partial(
        pl.kernel,
        out_type=jax.ShapeDtypeStruct((3 * b, f), jnp.float32),
        mesh=mesh,
        scratch_types=[
            pltpu.VMEM((3, lw), jnp.int32),
            pltpu.VMEM((3 * lw, f), jnp.float32),
            pltpu.SemaphoreType.DMA,
        ],
    )
    def gather_k(utab_ref, itab_ref, idx_ref, out_ref, idx_v, rows_v, sem):
        wid = lax.axis_index("s") * info.num_cores + lax.axis_index("c")
        pltpu.sync_copy(idx_ref.at[wid], idx_v)
        srcs = (utab_ref, itab_ref, itab_ref)
        cps = [
            pltpu.async_copy(
                srcs[j].at[idx_v.at[j]],
                rows_v.at[pl.ds(j * lw, lw)],
                sem,
            )
            for j in range(3)
        ]
        for c in cps:
            c.wait()
        for j in range(3):
            pltpu.sync_copy(
                rows_v.at[pl.ds(j * lw, lw)],
                out_ref.at[pl.ds(j * b + wid * lw, lw)],
            )

    return gather_k(utab, itab, idx3d)


def _loss_body(rows_ref, pi_ref, pj_ref, loss_ref, loss2_ref):
    b = pi_ref.shape[0]
    u = rows_ref[pl.ds(0, b), :]
    ie = rows_ref[pl.ds(b, b), :]
    je = rows_ref[pl.ds(2 * b, b), :]
    pi = jnp.sum(u * ie, axis=1)
    pj = jnp.sum(u * je, axis=1)
    pi_ref[...] = pi
    pj_ref[...] = pj
    d = pi - pj
    loss2 = jnp.mean(jnp.log(1.0 + jnp.exp(-d)))
    l2 = 0.0001 * jnp.sum(u * u + ie * ie + je * je, axis=1)
    loss2_ref[...] = jnp.reshape(loss2, (1, 1))
    loss_ref[...] = jnp.reshape(loss2 + jnp.mean(l2), (1, 1))


def _loss(rows):
    b = rows.shape[0] // 3
    return pl.pallas_call(
        _loss_body,
        out_shape=(
            jax.ShapeDtypeStruct((b,), jnp.float32),
            jax.ShapeDtypeStruct((b,), jnp.float32),
            jax.ShapeDtypeStruct((1, 1), jnp.float32),
            jax.ShapeDtypeStruct((1, 1), jnp.float32),
        ),
    )(rows)


@jax.jit
def kernel(user, item_i, item_j, user_item_3, item_user_3, user_js,
           embed_user_weight, embed_item_weight,
           user_item_matrix, item_user_matrix):
    ue = embed_user_weight
    ie = embed_item_weight
    nw = 32
    lw = user.shape[0] // nw

    g1i = _mm(item_user_matrix, ue)
    g1u, g2u = _mm2(user_item_matrix, ie.astype(jnp.bfloat16),
                    g1i.astype(jnp.bfloat16))
    gcn_users = _mm_users(user_item_3, ie, ue, g1u, g2u, user_js)
    g2i = _mm(item_user_matrix, g1u)
    gcn_items = _mm_items(item_user_3, ue, ie, g1i, g2i)

    idx = jnp.stack(
        [user.astype(jnp.int32).reshape(nw, lw),
         item_i.astype(jnp.int32).reshape(nw, lw),
         item_j.astype(jnp.int32).reshape(nw, lw)], axis=1)
    rows = _sc_gather(gcn_users, gcn_items, idx)

    pi, pj, loss, loss2 = _loss(rows)
    return pi, pj, loss[0, 0], loss2[0, 0]


# speedup vs baseline: 1.0190x; 1.0190x over previous
import functools

import jax
import jax.numpy as jnp
from jax import lax
from jax.experimental import pallas as pl
from jax.experimental.pallas import tpu as pltpu
from jax.experimental.pallas import tpu_sc as plsc


_BM = 256


def _mm_body(a_ref, x_ref, o_ref):
    o_ref[...] = jnp.dot(a_ref[...], x_ref[...],
                         preferred_element_type=jnp.float32)


def _mm(a, x, bm=_BM):
    m, k = a.shape
    n = x.shape[1]
    return pl.pallas_call(
        _mm_body,
        grid=(m // bm,),
        in_specs=[
            pl.BlockSpec((bm, k), lambda i: (i, 0)),
            pl.BlockSpec((k, n), lambda i: (0, 0)),
        ],
        out_specs=pl.BlockSpec((bm, n), lambda i: (i, 0)),
        out_shape=jax.ShapeDtypeStruct((m, n), jnp.float32),
    )(a, x)


def _mm2_body(a_ref, x1_ref, x2_ref, o1_ref, o2_ref):
    a = a_ref[...]
    o1_ref[...] = jnp.dot(a, x1_ref[...], preferred_element_type=jnp.float32)
    o2_ref[...] = jnp.dot(a, x2_ref[...], preferred_element_type=jnp.float32)


def _mm2(a, x1, x2, bm=_BM):
    m, k = a.shape
    n = x1.shape[1]
    return pl.pallas_call(
        _mm2_body,
        grid=(m // bm,),
        in_specs=[
            pl.BlockSpec((bm, k), lambda i: (i, 0)),
            pl.BlockSpec((k, n), lambda i: (0, 0)),
            pl.BlockSpec((k, n), lambda i: (0, 0)),
        ],
        out_specs=(
            pl.BlockSpec((bm, n), lambda i: (i, 0)),
            pl.BlockSpec((bm, n), lambda i: (i, 0)),
        ),
        out_shape=(
            jax.ShapeDtypeStruct((m, n), jnp.float32),
            jax.ShapeDtypeStruct((m, n), jnp.float32),
        ),
    )(a, x1, x2)


def _p12_body(a1_ref, a2_ref, ue_ref, ie_ref, g1i_ref, g1u_ref, g2u_ref,
              g1i_s):
    nb = pl.num_programs(0) // 2
    i = pl.program_id(0)
    bm = a1_ref.shape[0]

    @pl.when(i < nb)
    def _():
        blk = jnp.dot(a1_ref[...], ue_ref[...],
                      preferred_element_type=jnp.float32)
        g1i_ref[...] = blk
        g1i_s[pl.ds(i * bm, bm), :] = blk

    @pl.when(i >= nb)
    def _():
        a = a2_ref[...]
        g1u_ref[...] = jnp.dot(a, ie_ref[...],
                               preferred_element_type=jnp.float32)
        g2u_ref[...] = jnp.dot(a, g1i_s[...],
                               preferred_element_type=jnp.float32)


def _p12(iu, ui, ue, ie, bm=_BM):
    m, k = iu.shape
    n = ue.shape[1]
    nb = m // bm
    return pl.pallas_call(
        _p12_body,
        grid=(2 * nb,),
        in_specs=[
            pl.BlockSpec((bm, k), lambda i: (jnp.minimum(i, nb - 1), 0)),
            pl.BlockSpec((bm, k), lambda i: (jnp.maximum(i - nb, 0), 0)),
            pl.BlockSpec((k, n), lambda i: (0, 0)),
            pl.BlockSpec((k, n), lambda i: (0, 0)),
        ],
        out_specs=(
            pl.BlockSpec((bm, n), lambda i: (jnp.minimum(i, nb - 1), 0)),
            pl.BlockSpec((bm, n), lambda i: (jnp.maximum(i - nb, 0), 0)),
            pl.BlockSpec((bm, n), lambda i: (jnp.maximum(i - nb, 0), 0)),
        ),
        out_shape=(
            jax.ShapeDtypeStruct((m, n), jnp.float32),
            jax.ShapeDtypeStruct((m, n), jnp.float32),
            jax.ShapeDtypeStruct((m, n), jnp.float32),
        ),
        scratch_shapes=[pltpu.VMEM((k, n), jnp.float32)],
    )(iu, ui, ue, ie)


def _p45_body(a1_ref, a2_ref, g1u_ref, ue_ref, ie_ref, g1i_ref, o_ref,
              g2i_s):
    nb = pl.num_programs(0) // 2
    i = pl.program_id(0)
    bm = a1_ref.shape[0]

    @pl.when(i < nb)
    def _():
        g2i_s[pl.ds(i * bm, bm), :] = jnp.dot(
            a1_ref[...], g1u_ref[...], preferred_element_type=jnp.float32)

    @pl.when(i >= nb)
    def _():
        g3 = jnp.dot(a2_ref[...], ue_ref[...],
                     preferred_element_type=jnp.float32)
        o_ref[...] = 0.25 * (ie_ref[...] + g1i_ref[...]
                             + g2i_s[pl.ds((i - nb) * bm, bm), :] + g3)


def _p45(iu, iu3, g1u, ue, ie, g1i, bm=_BM):
    m, k = iu.shape
    n = ue.shape[1]
    nb = m // bm
    return pl.pallas_call(
        _p45_body,
        grid=(2 * nb,),
        in_specs=[
            pl.BlockSpec((bm, k), lambda i: (jnp.minimum(i, nb - 1), 0)),
            pl.BlockSpec((bm, k), lambda i: (jnp.maximum(i - nb, 0), 0)),
            pl.BlockSpec((k, n), lambda i: (0, 0)),
            pl.BlockSpec((k, n), lambda i: (0, 0)),
            pl.BlockSpec((bm, n), lambda i: (jnp.maximum(i - nb, 0), 0)),
            pl.BlockSpec((bm, n), lambda i: (jnp.maximum(i - nb, 0), 0)),
        ],
        out_specs=pl.BlockSpec((bm, n), lambda i: (jnp.maximum(i - nb, 0), 0)),
        out_shape=jax.ShapeDtypeStruct((m, n), jnp.float32),
        scratch_shapes=[pltpu.VMEM((m, n), jnp.float32)],
    )(iu, iu3, g1u, ue, ie, g1i)


def _mm_users_body(a_ref, x_ref, ue_ref, g1_ref, g2_ref, js_ref, o_ref):
    g3 = jnp.dot(a_ref[...], x_ref[...], preferred_element_type=jnp.float32)
    o_ref[...] = (0.25 * (ue_ref[...] + g1_ref[...] + g2_ref[...])
                  + g3 * js_ref[...])


def _mm_users(a, x, ue, g1, g2, js, bm=_BM):
    m, k = a.shape
    n = x.shape[1]
    return pl.pallas_call(
        _mm_users_body,
        grid=(m // bm,),
        in_specs=[
            pl.BlockSpec((bm, k), lambda i: (i, 0)),
            pl.BlockSpec((k, n), lambda i: (0, 0)),
            pl.BlockSpec((bm, n), lambda i: (i, 0)),
            pl.BlockSpec((bm, n), lambda i: (i, 0)),
            pl.BlockSpec((bm, n), lambda i: (i, 0)),
            pl.BlockSpec((bm, 1), lambda i: (i, 0)),
        ],
        out_specs=pl.BlockSpec((bm, n), lambda i: (i, 0)),
        out_shape=jax.ShapeDtypeStruct((m, n), jnp.float32),
    )(a, x, ue, g1, g2, js)


def _mm_items_body(a_ref, x_ref, ie_ref, g1_ref, g2_ref, o_ref):
    g3 = jnp.dot(a_ref[...], x_ref[...], preferred_element_type=jnp.float32)
    o_ref[...] = 0.25 * (ie_ref[...] + g1_ref[...] + g2_ref[...] + g3)


def _mm_items(a, x, ie, g1, g2, bm=_BM):
    m, k = a.shape
    n = x.shape[1]
    return pl.pallas_call(
        _mm_items_body,
        grid=(m // bm,),
        in_specs=[
            pl.BlockSpec((bm, k), lambda i: (i, 0)),
            pl.BlockSpec((k, n), lambda i: (0, 0)),
            pl.BlockSpec((bm, n), lambda i: (i, 0)),
            pl.BlockSpec((bm, n), lambda i: (i, 0)),
            pl.BlockSpec((bm, n), lambda i: (i, 0)),
        ],
        out_specs=pl.BlockSpec((bm, n), lambda i: (i, 0)),
        out_shape=jax.ShapeDtypeStruct((m, n), jnp.float32),
    )(a, x, ie, g1, g2)


def _sc_gather(utab, itab, idx3d):
    _, f = utab.shape
    nw, three, lw = idx3d.shape
    info = plsc.get_sparse_core_info()
    assert nw == info.num_cores * info.num_subcores and three == 3
    b = nw * lw
    mesh = plsc.VectorSubcoreMesh(core_axis_name="c", subcore_axis_name="s")

    @functools.partial(
        pl.kernel,
        out_type=jax.ShapeDtypeStruct((3 * b, f), jnp.float32),
        mesh=mesh,
        scratch_types=[
            pltpu.VMEM((3, lw), jnp.int32),
            pltpu.VMEM((3 * lw, f), jnp.float32),
            pltpu.SemaphoreType.DMA,
        ],
    )
    def gather_k(utab_ref, itab_ref, idx_ref, out_ref, idx_v, rows_v, sem):
        wid = lax.axis_index("s") * info.num_cores + lax.axis_index("c")
        pltpu.sync_copy(idx_ref.at[wid], idx_v)
        srcs = (utab_ref, itab_ref, itab_ref)
        cps = [
            pltpu.async_copy(
                srcs[j].at[idx_v.at[j]],
                rows_v.at[pl.ds(j * lw, lw)],
                sem,
            )
            for j in range(3)
        ]
        for c in cps:
            c.wait()
        for j in range(3):
            pltpu.sync_copy(
                rows_v.at[pl.ds(j * lw, lw)],
                out_ref.at[pl.ds(j * b + wid * lw, lw)],
            )

    return gather_k(utab, itab, idx3d)


def _loss_body(rows_ref, pi_ref, pj_ref, loss_ref, loss2_ref):
    b = pi_ref.shape[0]
    u = rows_ref[pl.ds(0, b), :]
    ie = rows_ref[pl.ds(b, b), :]
    je = rows_ref[pl.ds(2 * b, b), :]
    pi = jnp.sum(u * ie, axis=1)
    pj = jnp.sum(u * je, axis=1)
    pi_ref[...] = pi
    pj_ref[...] = pj
    d = pi - pj
    loss2 = jnp.mean(jnp.log(1.0 + jnp.exp(-d)))
    l2 = 0.0001 * jnp.sum(u * u + ie * ie + je * je, axis=1)
    loss2_ref[...] = jnp.reshape(loss2, (1, 1))
    loss_ref[...] = jnp.reshape(loss2 + jnp.mean(l2), (1, 1))


def _loss(rows):
    b = rows.shape[0] // 3
    return pl.pallas_call(
        _loss_body,
        out_shape=(
            jax.ShapeDtypeStruct((b,), jnp.float32),
            jax.ShapeDtypeStruct((b,), jnp.float32),
            jax.ShapeDtypeStruct((1, 1), jnp.float32),
            jax.ShapeDtypeStruct((1, 1), jnp.float32),
        ),
    )(rows)


@jax.jit
def kernel(user, item_i, item_j, user_item_3, item_user_3, user_js,
           embed_user_weight, embed_item_weight,
           user_item_matrix, item_user_matrix):
    ue = embed_user_weight
    ie = embed_item_weight
    nw = 32
    lw = user.shape[0] // nw

    g1i, g1u, g2u = _p12(item_user_matrix, user_item_matrix, ue, ie)
    gcn_users = _mm_users(user_item_3, ie, ue, g1u, g2u, user_js)
    gcn_items = _p45(item_user_matrix, item_user_3, g1u, ue, ie, g1i)

    idx = jnp.stack(
        [user.astype(jnp.int32).reshape(nw, lw),
         item_i.astype(jnp.int32).reshape(nw, lw),
         item_j.astype(jnp.int32).reshape(nw, lw)], axis=1)
    rows = _sc_gather(gcn_users, gcn_items, idx)

    pi, pj, loss, loss2 = _loss(rows)
    return pi, pj, loss[0, 0], loss2[0, 0]
